# Initial kernel scaffold; baseline (speedup 1.0000x reference)
import jax
import jax.numpy as jnp
from jax.experimental import pallas as pl


def kernel(positions, neighbors_j, neighbors_k):
    raise NotImplementedError("write your pallas kernel here")


# speedup vs baseline: 398.5297x; 398.5297x over previous
import functools

import jax
import jax.numpy as jnp
from jax import lax
from jax.experimental import pallas as pl
from jax.experimental.pallas import tpu as pltpu
from jax.experimental.pallas import tpu_sc as plsc

B, A, T = 16, 4096, 128
L = 16
NW = 32
ROWS_PER_W = (B * A) // NW
W_PER_B = A // ROWS_PER_W
CR = 64
NCHUNK = ROWS_PER_W // CR
NV = T // L


def _rsqrt(x):
    i = plsc.bitcast(x, jnp.int32)
    i = jnp.int32(0x5F3759DF) - (i >> 1)
    y = plsc.bitcast(i, jnp.float32)
    xh = x * jnp.float32(0.5)
    y = y * (jnp.float32(1.5) - xh * y * y)
    y = y * (jnp.float32(1.5) - xh * y * y)
    y = y * (jnp.float32(1.5) - xh * y * y)
    return y


def _sqrt(x):
    return x * _rsqrt(x)


def _body(pos_hbm, nj_hbm, nk_hbm, rij_hbm, rik_hbm, rjk_hbm,
          pos_v, nj_v, nk_v, rij_v, rik_v, rjk_v):
    c = lax.axis_index("c")
    s = lax.axis_index("s")
    wid = s * 2 + c
    b = wid // W_PER_B
    a_base = (wid % W_PER_B) * ROWS_PER_W

    pltpu.sync_copy(pos_hbm.at[b], pos_v)

    offy = jnp.full((L,), A, jnp.int32)
    offz = jnp.full((L,), 2 * A, jnp.int32)
    eps = jnp.float32(1e-9)

    def chunk_body(ci, _):
        a0 = a_base + ci * CR
        pltpu.sync_copy(nj_hbm.at[b, pl.ds(a0, CR)], nj_v)
        pltpu.sync_copy(nk_hbm.at[b, pl.ds(a0, CR)], nk_v)

        def row_body(r, _):
            ia = jnp.full((L,), a0 + r, jnp.int32)
            xi = plsc.load_gather(pos_v, [ia])
            yi = plsc.load_gather(pos_v, [ia + offy])
            zi = plsc.load_gather(pos_v, [ia + offz])
            for v in range(NV):
                idx_j = nj_v[r, pl.ds(v * L, L)]
                idx_k = nk_v[r, pl.ds(v * L, L)]
                xj = plsc.load_gather(pos_v, [idx_j])
                yj = plsc.load_gather(pos_v, [idx_j + offy])
                zj = plsc.load_gather(pos_v, [idx_j + offz])
                xk = plsc.load_gather(pos_v, [idx_k])
                yk = plsc.load_gather(pos_v, [idx_k + offy])
                zk = plsc.load_gather(pos_v, [idx_k + offz])
                dxij = xj - xi
                dyij = yj - yi
                dzij = zj - zi
                dxik = xk - xi
                dyik = yk - yi
                dzik = zk - zi
                dxjk = xj - xk
                dyjk = yj - yk
                dzjk = zj - zk
                rij = _sqrt(dxij * dxij + dyij * dyij + dzij * dzij) + eps
                rik = _sqrt(dxik * dxik + dyik * dyik + dzik * dzik) + eps
                rjk = _sqrt(dxjk * dxjk + dyjk * dyjk + dzjk * dzjk) + eps
                rij_v[r, pl.ds(v * L, L)] = rij
                rik_v[r, pl.ds(v * L, L)] = rik
                rjk_v[r, pl.ds(v * L, L)] = rjk
            return ()

        lax.fori_loop(0, CR, row_body, (), unroll=1)
        pltpu.sync_copy(rij_v, rij_hbm.at[b, pl.ds(a0, CR)])
        pltpu.sync_copy(rik_v, rik_hbm.at[b, pl.ds(a0, CR)])
        pltpu.sync_copy(rjk_v, rjk_hbm.at[b, pl.ds(a0, CR)])
        return ()

    lax.fori_loop(0, NCHUNK, chunk_body, (), unroll=1)


_triples = functools.partial(
    pl.kernel,
    out_type=(
        jax.ShapeDtypeStruct((B, A, T), jnp.float32),
        jax.ShapeDtypeStruct((B, A, T), jnp.float32),
        jax.ShapeDtypeStruct((B, A, T), jnp.float32),
    ),
    mesh=plsc.VectorSubcoreMesh(core_axis_name="c", subcore_axis_name="s"),
    compiler_params=pltpu.CompilerParams(needs_layout_passes=False),
    scratch_types=[
        pltpu.VMEM((3 * A,), jnp.float32),
        pltpu.VMEM((CR, T), jnp.int32),
        pltpu.VMEM((CR, T), jnp.int32),
        pltpu.VMEM((CR, T), jnp.float32),
        pltpu.VMEM((CR, T), jnp.float32),
        pltpu.VMEM((CR, T), jnp.float32),
    ],
)(_body)


def kernel(positions, neighbors_j, neighbors_k):
    nj = neighbors_j.astype(jnp.int32)
    nk = neighbors_k.astype(jnp.int32)
    pos_flat = positions.transpose(0, 2, 1).reshape(B, 3 * A)
    return _triples(pos_flat, nj, nk)
